# Initial kernel scaffold; baseline (speedup 1.0000x reference)
import jax
import jax.numpy as jnp
from jax.experimental import pallas as pl


def kernel(x, edge_index, W1, b1, W2, b2, W3, b3, Wr, br):
    raise NotImplementedError("write your pallas kernel here")


# speedup vs baseline: 10.7151x; 10.7151x over previous
import functools

import jax
import jax.numpy as jnp
from jax import lax
from jax.experimental import pallas as pl
from jax.experimental.pallas import tpu as pltpu
from jax.experimental.pallas import tpu_sc as plsc

NC, NS = 2, 16
NW = NC * NS
K = 80


def _flat_ids():
    sid = lax.axis_index("s")
    cid = lax.axis_index("c")
    return cid, sid, sid * NC + cid


def _make_deg(n, e):
    epw = e // NW
    chunk = 2000
    nchunk = epw // chunk
    mesh = plsc.VectorSubcoreMesh(core_axis_name="c", subcore_axis_name="s")

    @functools.partial(
        pl.kernel,
        out_type=jax.ShapeDtypeStruct((NW * n,), jnp.float32),
        mesh=mesh,
        scratch_types=[
            pltpu.VMEM((chunk,), jnp.int32),
            pltpu.VMEM((n,), jnp.float32),
        ],
        compiler_params=pltpu.CompilerParams(needs_layout_passes=False),
    )
    def deg_kernel(dst_hbm, zeros_hbm, out_hbm, dstv, degv):
        _, _, wid = _flat_ids()
        pltpu.sync_copy(zeros_hbm, degv)
        ones = jnp.ones((16,), jnp.float32)

        def body(c, carry):
            base = pl.multiple_of(wid * epw + c * chunk, 8)
            pltpu.sync_copy(dst_hbm.at[pl.ds(base, chunk)], dstv)
            for j in range(chunk // 16):
                idx = dstv[pl.ds(j * 16, 16)]
                plsc.addupdate_scatter(degv, [idx], ones)
            return carry

        lax.fori_loop(0, nchunk, body, 0)
        pltpu.sync_copy(degv, out_hbm.at[pl.ds(pl.multiple_of(wid * n, 8), n)])

    return deg_kernel


def _make_agg(n, e, d):
    epw = e // NW
    nchunk = epw // K
    rpt = ((n // NS) // 8) * 8
    last = n - rpt * (NS - 1)
    mesh = plsc.VectorSubcoreMesh(core_axis_name="c", subcore_axis_name="s")

    @functools.partial(
        pl.kernel,
        out_type=jax.ShapeDtypeStruct((NC, n, d), jnp.float32),
        mesh=mesh,
        scratch_types=[
            pltpu.VMEM((K,), jnp.int32),
            pltpu.VMEM((K,), jnp.int32),
            pltpu.VMEM((K, d), jnp.float32),
            pltpu.VMEM_SHARED((n, d), jnp.float32),
            pltpu.SemaphoreType.DMA,
        ],
    )
    def agg_kernel(table_hbm, src_hbm, dst_hbm, zeros_hbm, out_hbm,
                   srcv, dstv, rows, acc, sem):
        cid, sid, wid = _flat_ids()
        row0 = pl.multiple_of(sid * rpt, 8)

        @pl.when(sid < NS - 1)
        def _():
            pltpu.sync_copy(zeros_hbm.at[pl.ds(row0, rpt)],
                            acc.at[pl.ds(row0, rpt)])

        @pl.when(sid == NS - 1)
        def _():
            pltpu.sync_copy(zeros_hbm.at[pl.ds((NS - 1) * rpt, last)],
                            acc.at[pl.ds((NS - 1) * rpt, last)])

        plsc.subcore_barrier()

        def body(c, carry):
            base = pl.multiple_of(wid * epw + c * K, 8)
            pltpu.sync_copy(src_hbm.at[pl.ds(base, K)], srcv)
            pltpu.sync_copy(dst_hbm.at[pl.ds(base, K)], dstv)
            pltpu.async_copy(table_hbm.at[srcv], rows, sem).wait()
            pltpu.sync_copy(rows, acc.at[dstv], add=True)
            return carry

        lax.fori_loop(0, nchunk, body, 0)
        plsc.subcore_barrier()

        @pl.when(sid < NS - 1)
        def _():
            pltpu.sync_copy(acc.at[pl.ds(row0, rpt)],
                            out_hbm.at[cid, pl.ds(row0, rpt)])

        @pl.when(sid == NS - 1)
        def _():
            pltpu.sync_copy(acc.at[pl.ds((NS - 1) * rpt, last)],
                            out_hbm.at[cid, pl.ds((NS - 1) * rpt, last)])

    return agg_kernel


_R = 1000


def _tc1(degp, x):
    n, d = x.shape

    def body(degp_ref, x_ref, y0_ref, dis_ref):
        deg = jnp.sum(degp_ref[...], axis=0) + 1.0
        dis = lax.rsqrt(deg)[:, None]
        dis_ref[...] = dis
        y0_ref[...] = x_ref[...] * dis

    return pl.pallas_call(
        body,
        out_shape=[
            jax.ShapeDtypeStruct((n, d), jnp.float32),
            jax.ShapeDtypeStruct((n, 1), jnp.float32),
        ],
    )(degp, x)


def _tc2(p, y0, dis, W1, b1):
    n, d = y0.shape
    dh = W1.shape[1]

    def body(p_ref, y0_ref, dis_ref, w_ref, b_ref, ya_ref, yb_ref):
        t = (p_ref[0] + p_ref[1] + y0_ref[...]) * dis_ref[...]
        h = jnp.dot(t, w_ref[...], preferred_element_type=jnp.float32)
        h = jnp.maximum(h + b_ref[...], 0.0)
        y = h * dis_ref[...]
        ya_ref[...] = y[:, :d]
        yb_ref[...] = y[:, d:]

    return pl.pallas_call(
        body,
        grid=(n // _R,),
        in_specs=[
            pl.BlockSpec((NC, _R, d), lambda i: (0, i, 0)),
            pl.BlockSpec((_R, d), lambda i: (i, 0)),
            pl.BlockSpec((_R, 1), lambda i: (i, 0)),
            pl.BlockSpec((d, dh), lambda i: (0, 0)),
            pl.BlockSpec((1, dh), lambda i: (0, 0)),
        ],
        out_specs=[
            pl.BlockSpec((_R, d), lambda i: (i, 0)),
            pl.BlockSpec((_R, d), lambda i: (i, 0)),
        ],
        out_shape=[
            jax.ShapeDtypeStruct((n, d), jnp.float32),
            jax.ShapeDtypeStruct((n, d), jnp.float32),
        ],
    )(p, y0, dis, W1, b1)


def _tc3(pa, pb, y1a, y1b, dis, W2, b2, W3):
    n, d = y1a.shape
    dh = W2.shape[0]

    def body(pa_ref, pb_ref, ya_ref, yb_ref, dis_ref, w2_ref, b2_ref, w3_ref,
             q_ref):
        ta = (pa_ref[0] + pa_ref[1] + ya_ref[...]) * dis_ref[...]
        tb = (pb_ref[0] + pb_ref[1] + yb_ref[...]) * dis_ref[...]
        t = jnp.concatenate([ta, tb], axis=1)
        h2 = jnp.dot(t, w2_ref[...], preferred_element_type=jnp.float32)
        h2 = jnp.maximum(h2 + b2_ref[...], 0.0)
        q = jnp.dot(h2, w3_ref[...], preferred_element_type=jnp.float32)
        q_ref[...] = q * dis_ref[...]

    return pl.pallas_call(
        body,
        grid=(n // _R,),
        in_specs=[
            pl.BlockSpec((NC, _R, d), lambda i: (0, i, 0)),
            pl.BlockSpec((NC, _R, d), lambda i: (0, i, 0)),
            pl.BlockSpec((_R, d), lambda i: (i, 0)),
            pl.BlockSpec((_R, d), lambda i: (i, 0)),
            pl.BlockSpec((_R, 1), lambda i: (i, 0)),
            pl.BlockSpec((dh, dh), lambda i: (0, 0)),
            pl.BlockSpec((1, dh), lambda i: (0, 0)),
            pl.BlockSpec((dh, d), lambda i: (0, 0)),
        ],
        out_specs=pl.BlockSpec((_R, d), lambda i: (i, 0)),
        out_shape=jax.ShapeDtypeStruct((n, d), jnp.float32),
    )(pa, pb, y1a, y1b, dis, W2, b2, W3)


def _tc4(pq, q, dis, b3, Wr, br):
    n, d = q.shape
    do = Wr.shape[1]

    def body(pq_ref, q_ref, dis_ref, b3_ref, wr_ref, br_ref, h_ref, rec_ref):
        z = (pq_ref[0] + pq_ref[1] + q_ref[...]) * dis_ref[...] + b3_ref[...]
        h = jnp.maximum(z, 0.0)
        h_ref[...] = h
        rec = jnp.dot(h, wr_ref[...], preferred_element_type=jnp.float32)
        rec_ref[...] = rec + br_ref[...]

    return pl.pallas_call(
        body,
        grid=(n // _R,),
        in_specs=[
            pl.BlockSpec((NC, _R, d), lambda i: (0, i, 0)),
            pl.BlockSpec((_R, d), lambda i: (i, 0)),
            pl.BlockSpec((_R, 1), lambda i: (i, 0)),
            pl.BlockSpec((1, d), lambda i: (0, 0)),
            pl.BlockSpec((d, do), lambda i: (0, 0)),
            pl.BlockSpec((1, do), lambda i: (0, 0)),
        ],
        out_specs=[
            pl.BlockSpec((_R, d), lambda i: (i, 0)),
            pl.BlockSpec((_R, do), lambda i: (i, 0)),
        ],
        out_shape=[
            jax.ShapeDtypeStruct((n, d), jnp.float32),
            jax.ShapeDtypeStruct((n, do), jnp.float32),
        ],
    )(pq, q, dis, b3, Wr, br)


def kernel(x, edge_index, W1, b1, W2, b2, W3, b3, Wr, br):
    n, d_in = x.shape
    e = edge_index.shape[1]
    src = edge_index[0].astype(jnp.int32)
    dst = edge_index[1].astype(jnp.int32)
    zeros1d = jnp.zeros((n,), jnp.float32)
    zeros2d = jnp.zeros((n, d_in), jnp.float32)

    deg_fn = _make_deg(n, e)
    agg_fn = _make_agg(n, e, d_in)

    degp = deg_fn(dst, zeros1d).reshape(NW, n)
    y0, dis = _tc1(degp, x)
    p1 = agg_fn(y0, src, dst, zeros2d)
    y1a, y1b = _tc2(p1, y0, dis, W1, b1.reshape(1, -1))
    pa = agg_fn(y1a, src, dst, zeros2d)
    pb = agg_fn(y1b, src, dst, zeros2d)
    q = _tc3(pa, pb, y1a, y1b, dis, W2, b2.reshape(1, -1), W3)
    pq = agg_fn(q, src, dst, zeros2d)
    h, rec = _tc4(pq, q, dis, b3.reshape(1, -1), Wr, br.reshape(1, -1))
    return (h, rec)


# speedup vs baseline: 24.8694x; 2.3210x over previous
import functools

import jax
import jax.numpy as jnp
from jax import lax
from jax.experimental import pallas as pl
from jax.experimental.pallas import tpu as pltpu
from jax.experimental.pallas import tpu_sc as plsc

NC, NS = 2, 16
NW = NC * NS
K = 80


def _flat_ids():
    sid = lax.axis_index("s")
    cid = lax.axis_index("c")
    return cid, sid, sid * NC + cid


def _make_deg(n, e):
    epw = e // NW
    chunk = 2000
    nchunk = epw // chunk
    mesh = plsc.VectorSubcoreMesh(core_axis_name="c", subcore_axis_name="s")

    @functools.partial(
        pl.kernel,
        out_type=jax.ShapeDtypeStruct((NW * n,), jnp.float32),
        mesh=mesh,
        scratch_types=[
            pltpu.VMEM((chunk,), jnp.int32),
            pltpu.VMEM((n,), jnp.float32),
        ],
        compiler_params=pltpu.CompilerParams(needs_layout_passes=False),
    )
    def deg_kernel(dst_hbm, zeros_hbm, out_hbm, dstv, degv):
        _, _, wid = _flat_ids()
        pltpu.sync_copy(zeros_hbm, degv)
        ones = jnp.ones((16,), jnp.float32)

        def body(c, carry):
            base = pl.multiple_of(wid * epw + c * chunk, 8)
            pltpu.sync_copy(dst_hbm.at[pl.ds(base, chunk)], dstv)
            for j in range(chunk // 16):
                idx = dstv[pl.ds(j * 16, 16)]
                plsc.addupdate_scatter(degv, [idx], ones)
            return carry

        lax.fori_loop(0, nchunk, body, 0)
        pltpu.sync_copy(degv, out_hbm.at[pl.ds(pl.multiple_of(wid * n, 8), n)])

    return deg_kernel


def _make_agg(n, e, d):
    epw = e // NW
    nchunk = epw // K
    rpt = ((n // NS) // 8) * 8
    last = n - rpt * (NS - 1)
    mesh = plsc.VectorSubcoreMesh(core_axis_name="c", subcore_axis_name="s")

    @functools.partial(
        pl.kernel,
        out_type=jax.ShapeDtypeStruct((NC, n, d), jnp.float32),
        mesh=mesh,
        scratch_types=[
            pltpu.VMEM((epw,), jnp.int32),
            pltpu.VMEM((K,), jnp.int32),
            pltpu.VMEM((K,), jnp.int32),
            pltpu.VMEM((K, d), jnp.float32),
            pltpu.VMEM((K, d), jnp.float32),
            pltpu.VMEM_SHARED((n, d), jnp.float32),
            pltpu.SemaphoreType.DMA,
            pltpu.SemaphoreType.DMA,
        ],
    )
    def agg_kernel(table_hbm, src_hbm, dst_hbm, zeros_hbm, out_hbm,
                   srcall, dv0, dv1, r0, r1, acc, sem0, sem1):
        cid, sid, wid = _flat_ids()
        ebase = pl.multiple_of(wid * epw, 8)
        pltpu.sync_copy(src_hbm.at[pl.ds(ebase, epw)], srcall)
        row0 = pl.multiple_of(sid * rpt, 8)

        @pl.when(sid < NS - 1)
        def _():
            pltpu.sync_copy(zeros_hbm.at[pl.ds(row0, rpt)],
                            acc.at[pl.ds(row0, rpt)])

        @pl.when(sid == NS - 1)
        def _():
            pltpu.sync_copy(zeros_hbm.at[pl.ds((NS - 1) * rpt, last)],
                            acc.at[pl.ds((NS - 1) * rpt, last)])

        plsc.subcore_barrier()

        bufs = ((dv0, r0, sem0), (dv1, r1, sem1))

        def issue(c, par):
            dv, rv, sem = bufs[par]
            base = pl.multiple_of(wid * epw + c * K, 8)
            pltpu.async_copy(dst_hbm.at[pl.ds(base, K)], dv, sem)
            sl = pl.ds(pl.multiple_of(c * K, 8), K)
            pltpu.async_copy(table_hbm.at[srcall.at[sl]], rv, sem)

        issue(0, 0)

        def body(c, carry):
            for par in (0, 1):

                @pl.when((c & 1) == par)
                def _():
                    dv, rv, sem = bufs[par]

                    @pl.when(c + 1 < nchunk)
                    def _():
                        issue(c + 1, 1 - par)

                    pltpu.make_async_copy(
                        dst_hbm.at[pl.ds(0, K)], dv, sem).wait()
                    pltpu.make_async_copy(
                        table_hbm.at[pl.ds(0, K)], rv, sem).wait()
                    pltpu.sync_copy(rv, acc.at[dv], add=True)

            return carry

        lax.fori_loop(0, nchunk, body, 0)
        plsc.subcore_barrier()

        @pl.when(sid < NS - 1)
        def _():
            pltpu.sync_copy(acc.at[pl.ds(row0, rpt)],
                            out_hbm.at[cid, pl.ds(row0, rpt)])

        @pl.when(sid == NS - 1)
        def _():
            pltpu.sync_copy(acc.at[pl.ds((NS - 1) * rpt, last)],
                            out_hbm.at[cid, pl.ds((NS - 1) * rpt, last)])

    return agg_kernel


_R = 1000


def _tc1(degp, x):
    n, d = x.shape

    def body(degp_ref, x_ref, y0_ref, dis_ref):
        deg = jnp.sum(degp_ref[...], axis=0) + 1.0
        dis = lax.rsqrt(deg)[:, None]
        dis_ref[...] = dis
        y0_ref[...] = x_ref[...] * dis

    return pl.pallas_call(
        body,
        out_shape=[
            jax.ShapeDtypeStruct((n, d), jnp.float32),
            jax.ShapeDtypeStruct((n, 1), jnp.float32),
        ],
    )(degp, x)


def _tc2(p, y0, dis, W1, b1):
    n, d = y0.shape
    dh = W1.shape[1]

    def body(p_ref, y0_ref, dis_ref, w_ref, b_ref, ya_ref, yb_ref):
        t = (p_ref[0] + p_ref[1] + y0_ref[...]) * dis_ref[...]
        h = jnp.dot(t, w_ref[...], preferred_element_type=jnp.float32)
        h = jnp.maximum(h + b_ref[...], 0.0)
        y = h * dis_ref[...]
        ya_ref[...] = y[:, :d]
        yb_ref[...] = y[:, d:]

    return pl.pallas_call(
        body,
        grid=(n // _R,),
        in_specs=[
            pl.BlockSpec((NC, _R, d), lambda i: (0, i, 0)),
            pl.BlockSpec((_R, d), lambda i: (i, 0)),
            pl.BlockSpec((_R, 1), lambda i: (i, 0)),
            pl.BlockSpec((d, dh), lambda i: (0, 0)),
            pl.BlockSpec((1, dh), lambda i: (0, 0)),
        ],
        out_specs=[
            pl.BlockSpec((_R, d), lambda i: (i, 0)),
            pl.BlockSpec((_R, d), lambda i: (i, 0)),
        ],
        out_shape=[
            jax.ShapeDtypeStruct((n, d), jnp.float32),
            jax.ShapeDtypeStruct((n, d), jnp.float32),
        ],
    )(p, y0, dis, W1, b1)


def _tc3(pa, pb, y1a, y1b, dis, W2, b2, W3):
    n, d = y1a.shape
    dh = W2.shape[0]

    def body(pa_ref, pb_ref, ya_ref, yb_ref, dis_ref, w2_ref, b2_ref, w3_ref,
             q_ref):
        ta = (pa_ref[0] + pa_ref[1] + ya_ref[...]) * dis_ref[...]
        tb = (pb_ref[0] + pb_ref[1] + yb_ref[...]) * dis_ref[...]
        t = jnp.concatenate([ta, tb], axis=1)
        h2 = jnp.dot(t, w2_ref[...], preferred_element_type=jnp.float32)
        h2 = jnp.maximum(h2 + b2_ref[...], 0.0)
        q = jnp.dot(h2, w3_ref[...], preferred_element_type=jnp.float32)
        q_ref[...] = q * dis_ref[...]

    return pl.pallas_call(
        body,
        grid=(n // _R,),
        in_specs=[
            pl.BlockSpec((NC, _R, d), lambda i: (0, i, 0)),
            pl.BlockSpec((NC, _R, d), lambda i: (0, i, 0)),
            pl.BlockSpec((_R, d), lambda i: (i, 0)),
            pl.BlockSpec((_R, d), lambda i: (i, 0)),
            pl.BlockSpec((_R, 1), lambda i: (i, 0)),
            pl.BlockSpec((dh, dh), lambda i: (0, 0)),
            pl.BlockSpec((1, dh), lambda i: (0, 0)),
            pl.BlockSpec((dh, d), lambda i: (0, 0)),
        ],
        out_specs=pl.BlockSpec((_R, d), lambda i: (i, 0)),
        out_shape=jax.ShapeDtypeStruct((n, d), jnp.float32),
    )(pa, pb, y1a, y1b, dis, W2, b2, W3)


def _tc4(pq, q, dis, b3, Wr, br):
    n, d = q.shape
    do = Wr.shape[1]

    def body(pq_ref, q_ref, dis_ref, b3_ref, wr_ref, br_ref, h_ref, rec_ref):
        z = (pq_ref[0] + pq_ref[1] + q_ref[...]) * dis_ref[...] + b3_ref[...]
        h = jnp.maximum(z, 0.0)
        h_ref[...] = h
        rec = jnp.dot(h, wr_ref[...], preferred_element_type=jnp.float32)
        rec_ref[...] = rec + br_ref[...]

    return pl.pallas_call(
        body,
        grid=(n // _R,),
        in_specs=[
            pl.BlockSpec((NC, _R, d), lambda i: (0, i, 0)),
            pl.BlockSpec((_R, d), lambda i: (i, 0)),
            pl.BlockSpec((_R, 1), lambda i: (i, 0)),
            pl.BlockSpec((1, d), lambda i: (0, 0)),
            pl.BlockSpec((d, do), lambda i: (0, 0)),
            pl.BlockSpec((1, do), lambda i: (0, 0)),
        ],
        out_specs=[
            pl.BlockSpec((_R, d), lambda i: (i, 0)),
            pl.BlockSpec((_R, do), lambda i: (i, 0)),
        ],
        out_shape=[
            jax.ShapeDtypeStruct((n, d), jnp.float32),
            jax.ShapeDtypeStruct((n, do), jnp.float32),
        ],
    )(pq, q, dis, b3, Wr, br)


def kernel(x, edge_index, W1, b1, W2, b2, W3, b3, Wr, br):
    n, d_in = x.shape
    e = edge_index.shape[1]
    src = edge_index[0].astype(jnp.int32)
    dst = edge_index[1].astype(jnp.int32)
    zeros1d = jnp.zeros((n,), jnp.float32)
    zeros2d = jnp.zeros((n, d_in), jnp.float32)

    deg_fn = _make_deg(n, e)
    agg_fn = _make_agg(n, e, d_in)

    degp = deg_fn(dst, zeros1d).reshape(NW, n)
    y0, dis = _tc1(degp, x)
    p1 = agg_fn(y0, src, dst, zeros2d)
    y1a, y1b = _tc2(p1, y0, dis, W1, b1.reshape(1, -1))
    pa = agg_fn(y1a, src, dst, zeros2d)
    pb = agg_fn(y1b, src, dst, zeros2d)
    q = _tc3(pa, pb, y1a, y1b, dis, W2, b2.reshape(1, -1), W3)
    pq = agg_fn(q, src, dst, zeros2d)
    h, rec = _tc4(pq, q, dis, b3.reshape(1, -1), Wr, br.reshape(1, -1))
    return (h, rec)


# speedup vs baseline: 28.5616x; 1.1485x over previous
import functools

import jax
import jax.numpy as jnp
from jax import lax
from jax.experimental import pallas as pl
from jax.experimental.pallas import tpu as pltpu
from jax.experimental.pallas import tpu_sc as plsc

NC, NS = 2, 16
NW = NC * NS
K = 80


def _flat_ids():
    sid = lax.axis_index("s")
    cid = lax.axis_index("c")
    return cid, sid, sid * NC + cid


def _make_deg(n, e):
    epw = e // NW
    chunk = 2000
    nchunk = epw // chunk
    mesh = plsc.VectorSubcoreMesh(core_axis_name="c", subcore_axis_name="s")

    @functools.partial(
        pl.kernel,
        out_type=jax.ShapeDtypeStruct((NW * n,), jnp.float32),
        mesh=mesh,
        scratch_types=[
            pltpu.VMEM((chunk,), jnp.int32),
            pltpu.VMEM((n,), jnp.float32),
        ],
        compiler_params=pltpu.CompilerParams(needs_layout_passes=False),
    )
    def deg_kernel(dst_hbm, zeros_hbm, out_hbm, dstv, degv):
        _, _, wid = _flat_ids()
        pltpu.sync_copy(zeros_hbm, degv)
        ones = jnp.ones((16,), jnp.float32)

        def body(c, carry):
            base = pl.multiple_of(wid * epw + c * chunk, 8)
            pltpu.sync_copy(dst_hbm.at[pl.ds(base, chunk)], dstv)
            for j in range(chunk // 16):
                idx = dstv[pl.ds(j * 16, 16)]
                plsc.addupdate_scatter(degv, [idx], ones)
            return carry

        lax.fori_loop(0, nchunk, body, 0)
        pltpu.sync_copy(degv, out_hbm.at[pl.ds(pl.multiple_of(wid * n, 8), n)])

    return deg_kernel


def _make_agg(n, e, d):
    epw = e // NW
    nchunk = epw // K
    rpt = ((n // NS) // 8) * 8
    last = n - rpt * (NS - 1)
    mesh = plsc.VectorSubcoreMesh(core_axis_name="c", subcore_axis_name="s")

    nbuf = 3

    @functools.partial(
        pl.kernel,
        out_type=jax.ShapeDtypeStruct((NC, n, d), jnp.float32),
        mesh=mesh,
        scratch_types=[
            pltpu.VMEM((epw,), jnp.int32),
            [pltpu.VMEM((K,), jnp.int32) for _ in range(nbuf)],
            [pltpu.VMEM((K, d), jnp.float32) for _ in range(nbuf)],
            pltpu.VMEM_SHARED((n, d), jnp.float32),
            [pltpu.SemaphoreType.DMA for _ in range(nbuf)],
            [pltpu.SemaphoreType.DMA for _ in range(nbuf)],
        ],
    )
    def agg_kernel(table_hbm, src_hbm, dst_hbm, zeros_hbm, out_hbm,
                   srcall, dvs, rvs, acc, gsems, ssems):
        cid, sid, wid = _flat_ids()
        ebase = pl.multiple_of(wid * epw, 8)
        pltpu.sync_copy(src_hbm.at[pl.ds(ebase, epw)], srcall)
        row0 = pl.multiple_of(sid * rpt, 8)

        @pl.when(sid < NS - 1)
        def _():
            pltpu.sync_copy(zeros_hbm.at[pl.ds(row0, rpt)],
                            acc.at[pl.ds(row0, rpt)])

        @pl.when(sid == NS - 1)
        def _():
            pltpu.sync_copy(zeros_hbm.at[pl.ds((NS - 1) * rpt, last)],
                            acc.at[pl.ds((NS - 1) * rpt, last)])

        plsc.subcore_barrier()

        def issue(c, b):
            base = pl.multiple_of(wid * epw + c * K, 8)
            pltpu.async_copy(dst_hbm.at[pl.ds(base, K)], dvs[b], gsems[b])
            sl = pl.ds(pl.multiple_of(c * K, 8), K)
            pltpu.async_copy(table_hbm.at[srcall.at[sl]], rvs[b], gsems[b])

        def wait_gather(b):
            pltpu.make_async_copy(dst_hbm.at[pl.ds(0, K)],
                                  dvs[b], gsems[b]).wait()
            pltpu.make_async_copy(table_hbm.at[pl.ds(0, K)],
                                  rvs[b], gsems[b]).wait()

        def wait_scatter(b):
            pltpu.make_async_copy(table_hbm.at[pl.ds(0, K)],
                                  rvs[b], ssems[b]).wait()

        for b in range(nbuf - 1):
            issue(b, b)

        def body(c, carry):
            for par in range(nbuf):

                @pl.when(lax.rem(c, nbuf) == par)
                def _():
                    wait_gather(par)
                    pltpu.async_copy(rvs[par], acc.at[dvs[par]],
                                     ssems[par], add=True)
                    nb = (par + nbuf - 1) % nbuf

                    @pl.when(c + nbuf - 1 < nchunk)
                    def _():
                        @pl.when(c >= 1)
                        def _():
                            wait_scatter(nb)

                        issue(c + nbuf - 1, nb)

            return carry

        lax.fori_loop(0, nchunk, body, 0)
        for b in range(nbuf):
            wait_scatter(b)
        plsc.subcore_barrier()

        @pl.when(sid < NS - 1)
        def _():
            pltpu.sync_copy(acc.at[pl.ds(row0, rpt)],
                            out_hbm.at[cid, pl.ds(row0, rpt)])

        @pl.when(sid == NS - 1)
        def _():
            pltpu.sync_copy(acc.at[pl.ds((NS - 1) * rpt, last)],
                            out_hbm.at[cid, pl.ds((NS - 1) * rpt, last)])

    return agg_kernel


_R = 1000


def _tc1(degp, x):
    n, d = x.shape

    def body(degp_ref, x_ref, y0_ref, dis_ref):
        deg = jnp.sum(degp_ref[...], axis=0) + 1.0
        dis = lax.rsqrt(deg)[:, None]
        dis_ref[...] = dis
        y0_ref[...] = x_ref[...] * dis

    return pl.pallas_call(
        body,
        out_shape=[
            jax.ShapeDtypeStruct((n, d), jnp.float32),
            jax.ShapeDtypeStruct((n, 1), jnp.float32),
        ],
    )(degp, x)


def _tc2(p, y0, dis, W1, b1):
    n, d = y0.shape
    dh = W1.shape[1]

    def body(p_ref, y0_ref, dis_ref, w_ref, b_ref, ya_ref, yb_ref):
        t = (p_ref[0] + p_ref[1] + y0_ref[...]) * dis_ref[...]
        h = jnp.dot(t, w_ref[...], preferred_element_type=jnp.float32)
        h = jnp.maximum(h + b_ref[...], 0.0)
        y = h * dis_ref[...]
        ya_ref[...] = y[:, :d]
        yb_ref[...] = y[:, d:]

    return pl.pallas_call(
        body,
        grid=(n // _R,),
        in_specs=[
            pl.BlockSpec((NC, _R, d), lambda i: (0, i, 0)),
            pl.BlockSpec((_R, d), lambda i: (i, 0)),
            pl.BlockSpec((_R, 1), lambda i: (i, 0)),
            pl.BlockSpec((d, dh), lambda i: (0, 0)),
            pl.BlockSpec((1, dh), lambda i: (0, 0)),
        ],
        out_specs=[
            pl.BlockSpec((_R, d), lambda i: (i, 0)),
            pl.BlockSpec((_R, d), lambda i: (i, 0)),
        ],
        out_shape=[
            jax.ShapeDtypeStruct((n, d), jnp.float32),
            jax.ShapeDtypeStruct((n, d), jnp.float32),
        ],
    )(p, y0, dis, W1, b1)


def _tc3(pa, pb, y1a, y1b, dis, W2, b2, W3):
    n, d = y1a.shape
    dh = W2.shape[0]

    def body(pa_ref, pb_ref, ya_ref, yb_ref, dis_ref, w2_ref, b2_ref, w3_ref,
             q_ref):
        ta = (pa_ref[0] + pa_ref[1] + ya_ref[...]) * dis_ref[...]
        tb = (pb_ref[0] + pb_ref[1] + yb_ref[...]) * dis_ref[...]
        t = jnp.concatenate([ta, tb], axis=1)
        h2 = jnp.dot(t, w2_ref[...], preferred_element_type=jnp.float32)
        h2 = jnp.maximum(h2 + b2_ref[...], 0.0)
        q = jnp.dot(h2, w3_ref[...], preferred_element_type=jnp.float32)
        q_ref[...] = q * dis_ref[...]

    return pl.pallas_call(
        body,
        grid=(n // _R,),
        in_specs=[
            pl.BlockSpec((NC, _R, d), lambda i: (0, i, 0)),
            pl.BlockSpec((NC, _R, d), lambda i: (0, i, 0)),
            pl.BlockSpec((_R, d), lambda i: (i, 0)),
            pl.BlockSpec((_R, d), lambda i: (i, 0)),
            pl.BlockSpec((_R, 1), lambda i: (i, 0)),
            pl.BlockSpec((dh, dh), lambda i: (0, 0)),
            pl.BlockSpec((1, dh), lambda i: (0, 0)),
            pl.BlockSpec((dh, d), lambda i: (0, 0)),
        ],
        out_specs=pl.BlockSpec((_R, d), lambda i: (i, 0)),
        out_shape=jax.ShapeDtypeStruct((n, d), jnp.float32),
    )(pa, pb, y1a, y1b, dis, W2, b2, W3)


def _tc4(pq, q, dis, b3, Wr, br):
    n, d = q.shape
    do = Wr.shape[1]

    def body(pq_ref, q_ref, dis_ref, b3_ref, wr_ref, br_ref, h_ref, rec_ref):
        z = (pq_ref[0] + pq_ref[1] + q_ref[...]) * dis_ref[...] + b3_ref[...]
        h = jnp.maximum(z, 0.0)
        h_ref[...] = h
        rec = jnp.dot(h, wr_ref[...], preferred_element_type=jnp.float32)
        rec_ref[...] = rec + br_ref[...]

    return pl.pallas_call(
        body,
        grid=(n // _R,),
        in_specs=[
            pl.BlockSpec((NC, _R, d), lambda i: (0, i, 0)),
            pl.BlockSpec((_R, d), lambda i: (i, 0)),
            pl.BlockSpec((_R, 1), lambda i: (i, 0)),
            pl.BlockSpec((1, d), lambda i: (0, 0)),
            pl.BlockSpec((d, do), lambda i: (0, 0)),
            pl.BlockSpec((1, do), lambda i: (0, 0)),
        ],
        out_specs=[
            pl.BlockSpec((_R, d), lambda i: (i, 0)),
            pl.BlockSpec((_R, do), lambda i: (i, 0)),
        ],
        out_shape=[
            jax.ShapeDtypeStruct((n, d), jnp.float32),
            jax.ShapeDtypeStruct((n, do), jnp.float32),
        ],
    )(pq, q, dis, b3, Wr, br)


def kernel(x, edge_index, W1, b1, W2, b2, W3, b3, Wr, br):
    n, d_in = x.shape
    e = edge_index.shape[1]
    src = edge_index[0].astype(jnp.int32)
    dst = edge_index[1].astype(jnp.int32)
    zeros1d = jnp.zeros((n,), jnp.float32)
    zeros2d = jnp.zeros((n, d_in), jnp.float32)

    deg_fn = _make_deg(n, e)
    agg_fn = _make_agg(n, e, d_in)

    degp = deg_fn(dst, zeros1d).reshape(NW, n)
    y0, dis = _tc1(degp, x)
    p1 = agg_fn(y0, src, dst, zeros2d)
    y1a, y1b = _tc2(p1, y0, dis, W1, b1.reshape(1, -1))
    pa = agg_fn(y1a, src, dst, zeros2d)
    pb = agg_fn(y1b, src, dst, zeros2d)
    q = _tc3(pa, pb, y1a, y1b, dis, W2, b2.reshape(1, -1), W3)
    pq = agg_fn(q, src, dst, zeros2d)
    h, rec = _tc4(pq, q, dis, b3.reshape(1, -1), Wr, br.reshape(1, -1))
    return (h, rec)


# speedup vs baseline: 30.2380x; 1.0587x over previous
import functools

import jax
import jax.numpy as jnp
from jax import lax
from jax.experimental import pallas as pl
from jax.experimental.pallas import tpu as pltpu
from jax.experimental.pallas import tpu_sc as plsc

NC, NS = 2, 16
NW = NC * NS
K = 40


def _flat_ids():
    sid = lax.axis_index("s")
    cid = lax.axis_index("c")
    return cid, sid, sid * NC + cid


def _make_deg(n, e):
    epw = e // NW
    chunk = 2000
    nchunk = epw // chunk
    mesh = plsc.VectorSubcoreMesh(core_axis_name="c", subcore_axis_name="s")

    @functools.partial(
        pl.kernel,
        out_type=jax.ShapeDtypeStruct((NW * n,), jnp.float32),
        mesh=mesh,
        scratch_types=[
            pltpu.VMEM((chunk,), jnp.int32),
            pltpu.VMEM((n,), jnp.float32),
        ],
        compiler_params=pltpu.CompilerParams(needs_layout_passes=False),
    )
    def deg_kernel(dst_hbm, zeros_hbm, out_hbm, dstv, degv):
        _, _, wid = _flat_ids()
        pltpu.sync_copy(zeros_hbm, degv)
        ones = jnp.ones((16,), jnp.float32)

        def body(c, carry):
            base = pl.multiple_of(wid * epw + c * chunk, 8)
            pltpu.sync_copy(dst_hbm.at[pl.ds(base, chunk)], dstv)
            for j in range(chunk // 16):
                idx = dstv[pl.ds(j * 16, 16)]
                plsc.addupdate_scatter(degv, [idx], ones)
            return carry

        lax.fori_loop(0, nchunk, body, 0)
        pltpu.sync_copy(degv, out_hbm.at[pl.ds(pl.multiple_of(wid * n, 8), n)])

    return deg_kernel


def _make_agg(n, e, d):
    epw = e // NW
    nchunk = epw // K
    rpt = ((n // NS) // 8) * 8
    last = n - rpt * (NS - 1)
    mesh = plsc.VectorSubcoreMesh(core_axis_name="c", subcore_axis_name="s")

    nbuf = 6

    @functools.partial(
        pl.kernel,
        out_type=jax.ShapeDtypeStruct((NC, n, d), jnp.float32),
        mesh=mesh,
        scratch_types=[
            pltpu.VMEM((epw,), jnp.int32),
            [pltpu.VMEM((K,), jnp.int32) for _ in range(nbuf)],
            [pltpu.VMEM((K, d), jnp.float32) for _ in range(nbuf)],
            pltpu.VMEM_SHARED((n, d), jnp.float32),
            [pltpu.SemaphoreType.DMA for _ in range(nbuf)],
            [pltpu.SemaphoreType.DMA for _ in range(nbuf)],
        ],
    )
    def agg_kernel(table_hbm, src_hbm, dst_hbm, zeros_hbm, out_hbm,
                   srcall, dvs, rvs, acc, gsems, ssems):
        cid, sid, wid = _flat_ids()
        ebase = pl.multiple_of(wid * epw, 8)
        pltpu.sync_copy(src_hbm.at[pl.ds(ebase, epw)], srcall)
        row0 = pl.multiple_of(sid * rpt, 8)

        @pl.when(sid < NS - 1)
        def _():
            pltpu.sync_copy(zeros_hbm.at[pl.ds(row0, rpt)],
                            acc.at[pl.ds(row0, rpt)])

        @pl.when(sid == NS - 1)
        def _():
            pltpu.sync_copy(zeros_hbm.at[pl.ds((NS - 1) * rpt, last)],
                            acc.at[pl.ds((NS - 1) * rpt, last)])

        plsc.subcore_barrier()

        def issue(c, b):
            base = pl.multiple_of(wid * epw + c * K, 8)
            pltpu.async_copy(dst_hbm.at[pl.ds(base, K)], dvs[b], gsems[b])
            sl = pl.ds(pl.multiple_of(c * K, 8), K)
            pltpu.async_copy(table_hbm.at[srcall.at[sl]], rvs[b], gsems[b])

        def wait_gather(b):
            pltpu.make_async_copy(dst_hbm.at[pl.ds(0, K)],
                                  dvs[b], gsems[b]).wait()
            pltpu.make_async_copy(table_hbm.at[pl.ds(0, K)],
                                  rvs[b], gsems[b]).wait()

        def wait_scatter(b):
            pltpu.make_async_copy(table_hbm.at[pl.ds(0, K)],
                                  rvs[b], ssems[b]).wait()

        for b in range(nbuf - 1):
            issue(b, b)

        def body(c, carry):
            for par in range(nbuf):

                @pl.when(lax.rem(c, nbuf) == par)
                def _():
                    wait_gather(par)
                    pltpu.async_copy(rvs[par], acc.at[dvs[par]],
                                     ssems[par], add=True)
                    nb = (par + nbuf - 1) % nbuf

                    @pl.when(c + nbuf - 1 < nchunk)
                    def _():
                        @pl.when(c >= 1)
                        def _():
                            wait_scatter(nb)

                        issue(c + nbuf - 1, nb)

            return carry

        lax.fori_loop(0, nchunk, body, 0)
        for b in range(nbuf):
            wait_scatter(b)
        plsc.subcore_barrier()

        @pl.when(sid < NS - 1)
        def _():
            pltpu.sync_copy(acc.at[pl.ds(row0, rpt)],
                            out_hbm.at[cid, pl.ds(row0, rpt)])

        @pl.when(sid == NS - 1)
        def _():
            pltpu.sync_copy(acc.at[pl.ds((NS - 1) * rpt, last)],
                            out_hbm.at[cid, pl.ds((NS - 1) * rpt, last)])

    return agg_kernel


_R = 1000


def _tc1(degp, x):
    n, d = x.shape

    def body(degp_ref, x_ref, y0_ref, dis_ref):
        deg = jnp.sum(degp_ref[...], axis=0) + 1.0
        dis = lax.rsqrt(deg)[:, None]
        dis_ref[...] = dis
        y0_ref[...] = x_ref[...] * dis

    return pl.pallas_call(
        body,
        out_shape=[
            jax.ShapeDtypeStruct((n, d), jnp.float32),
            jax.ShapeDtypeStruct((n, 1), jnp.float32),
        ],
    )(degp, x)


def _tc2(p, y0, dis, W1, b1):
    n, d = y0.shape
    dh = W1.shape[1]

    def body(p_ref, y0_ref, dis_ref, w_ref, b_ref, ya_ref, yb_ref):
        t = (p_ref[0] + p_ref[1] + y0_ref[...]) * dis_ref[...]
        h = jnp.dot(t, w_ref[...], preferred_element_type=jnp.float32)
        h = jnp.maximum(h + b_ref[...], 0.0)
        y = h * dis_ref[...]
        ya_ref[...] = y[:, :d]
        yb_ref[...] = y[:, d:]

    return pl.pallas_call(
        body,
        grid=(n // _R,),
        in_specs=[
            pl.BlockSpec((NC, _R, d), lambda i: (0, i, 0)),
            pl.BlockSpec((_R, d), lambda i: (i, 0)),
            pl.BlockSpec((_R, 1), lambda i: (i, 0)),
            pl.BlockSpec((d, dh), lambda i: (0, 0)),
            pl.BlockSpec((1, dh), lambda i: (0, 0)),
        ],
        out_specs=[
            pl.BlockSpec((_R, d), lambda i: (i, 0)),
            pl.BlockSpec((_R, d), lambda i: (i, 0)),
        ],
        out_shape=[
            jax.ShapeDtypeStruct((n, d), jnp.float32),
            jax.ShapeDtypeStruct((n, d), jnp.float32),
        ],
    )(p, y0, dis, W1, b1)


def _tc3(pa, pb, y1a, y1b, dis, W2, b2, W3):
    n, d = y1a.shape
    dh = W2.shape[0]

    def body(pa_ref, pb_ref, ya_ref, yb_ref, dis_ref, w2_ref, b2_ref, w3_ref,
             q_ref):
        ta = (pa_ref[0] + pa_ref[1] + ya_ref[...]) * dis_ref[...]
        tb = (pb_ref[0] + pb_ref[1] + yb_ref[...]) * dis_ref[...]
        t = jnp.concatenate([ta, tb], axis=1)
        h2 = jnp.dot(t, w2_ref[...], preferred_element_type=jnp.float32)
        h2 = jnp.maximum(h2 + b2_ref[...], 0.0)
        q = jnp.dot(h2, w3_ref[...], preferred_element_type=jnp.float32)
        q_ref[...] = q * dis_ref[...]

    return pl.pallas_call(
        body,
        grid=(n // _R,),
        in_specs=[
            pl.BlockSpec((NC, _R, d), lambda i: (0, i, 0)),
            pl.BlockSpec((NC, _R, d), lambda i: (0, i, 0)),
            pl.BlockSpec((_R, d), lambda i: (i, 0)),
            pl.BlockSpec((_R, d), lambda i: (i, 0)),
            pl.BlockSpec((_R, 1), lambda i: (i, 0)),
            pl.BlockSpec((dh, dh), lambda i: (0, 0)),
            pl.BlockSpec((1, dh), lambda i: (0, 0)),
            pl.BlockSpec((dh, d), lambda i: (0, 0)),
        ],
        out_specs=pl.BlockSpec((_R, d), lambda i: (i, 0)),
        out_shape=jax.ShapeDtypeStruct((n, d), jnp.float32),
    )(pa, pb, y1a, y1b, dis, W2, b2, W3)


def _tc4(pq, q, dis, b3, Wr, br):
    n, d = q.shape
    do = Wr.shape[1]

    def body(pq_ref, q_ref, dis_ref, b3_ref, wr_ref, br_ref, h_ref, rec_ref):
        z = (pq_ref[0] + pq_ref[1] + q_ref[...]) * dis_ref[...] + b3_ref[...]
        h = jnp.maximum(z, 0.0)
        h_ref[...] = h
        rec = jnp.dot(h, wr_ref[...], preferred_element_type=jnp.float32)
        rec_ref[...] = rec + br_ref[...]

    return pl.pallas_call(
        body,
        grid=(n // _R,),
        in_specs=[
            pl.BlockSpec((NC, _R, d), lambda i: (0, i, 0)),
            pl.BlockSpec((_R, d), lambda i: (i, 0)),
            pl.BlockSpec((_R, 1), lambda i: (i, 0)),
            pl.BlockSpec((1, d), lambda i: (0, 0)),
            pl.BlockSpec((d, do), lambda i: (0, 0)),
            pl.BlockSpec((1, do), lambda i: (0, 0)),
        ],
        out_specs=[
            pl.BlockSpec((_R, d), lambda i: (i, 0)),
            pl.BlockSpec((_R, do), lambda i: (i, 0)),
        ],
        out_shape=[
            jax.ShapeDtypeStruct((n, d), jnp.float32),
            jax.ShapeDtypeStruct((n, do), jnp.float32),
        ],
    )(pq, q, dis, b3, Wr, br)


def kernel(x, edge_index, W1, b1, W2, b2, W3, b3, Wr, br):
    n, d_in = x.shape
    e = edge_index.shape[1]
    src = edge_index[0].astype(jnp.int32)
    dst = edge_index[1].astype(jnp.int32)
    zeros1d = jnp.zeros((n,), jnp.float32)
    zeros2d = jnp.zeros((n, d_in), jnp.float32)

    deg_fn = _make_deg(n, e)
    agg_fn = _make_agg(n, e, d_in)

    degp = deg_fn(dst, zeros1d).reshape(NW, n)
    y0, dis = _tc1(degp, x)
    p1 = agg_fn(y0, src, dst, zeros2d)
    y1a, y1b = _tc2(p1, y0, dis, W1, b1.reshape(1, -1))
    pa = agg_fn(y1a, src, dst, zeros2d)
    pb = agg_fn(y1b, src, dst, zeros2d)
    q = _tc3(pa, pb, y1a, y1b, dis, W2, b2.reshape(1, -1), W3)
    pq = agg_fn(q, src, dst, zeros2d)
    h, rec = _tc4(pq, q, dis, b3.reshape(1, -1), Wr, br.reshape(1, -1))
    return (h, rec)


# speedup vs baseline: 30.5020x; 1.0087x over previous
import functools

import jax
import jax.numpy as jnp
from jax import lax
from jax.experimental import pallas as pl
from jax.experimental.pallas import tpu as pltpu
from jax.experimental.pallas import tpu_sc as plsc

NC, NS = 2, 16
NW = NC * NS
K = 40


def _flat_ids():
    sid = lax.axis_index("s")
    cid = lax.axis_index("c")
    return cid, sid, sid * NC + cid


def _make_deg(n, e):
    epw = e // NW
    chunk = 2000
    nchunk = epw // chunk
    mesh = plsc.VectorSubcoreMesh(core_axis_name="c", subcore_axis_name="s")

    @functools.partial(
        pl.kernel,
        out_type=jax.ShapeDtypeStruct((NW * n,), jnp.float32),
        mesh=mesh,
        scratch_types=[
            pltpu.VMEM((chunk,), jnp.int32),
            pltpu.VMEM((n,), jnp.float32),
        ],
        compiler_params=pltpu.CompilerParams(needs_layout_passes=False),
    )
    def deg_kernel(dst_hbm, zeros_hbm, out_hbm, dstv, degv):
        _, _, wid = _flat_ids()
        pltpu.sync_copy(zeros_hbm, degv)
        ones = jnp.ones((16,), jnp.float32)

        def body(c, carry):
            base = pl.multiple_of(wid * epw + c * chunk, 8)
            pltpu.sync_copy(dst_hbm.at[pl.ds(base, chunk)], dstv)
            for j in range(chunk // 16):
                idx = dstv[pl.ds(j * 16, 16)]
                plsc.addupdate_scatter(degv, [idx], ones)
            return carry

        lax.fori_loop(0, nchunk, body, 0)
        pltpu.sync_copy(degv, out_hbm.at[pl.ds(pl.multiple_of(wid * n, 8), n)])

    return deg_kernel


def _make_agg(n, e, d):
    epw = e // NW
    nchunk = epw // K
    rpt = ((n // NS) // 8) * 8
    last = n - rpt * (NS - 1)
    mesh = plsc.VectorSubcoreMesh(core_axis_name="c", subcore_axis_name="s")

    nbuf = 7

    @functools.partial(
        pl.kernel,
        out_type=jax.ShapeDtypeStruct((NC, n, d), jnp.float32),
        mesh=mesh,
        scratch_types=[
            pltpu.VMEM((epw,), jnp.int32),
            [pltpu.VMEM((K,), jnp.int32) for _ in range(nbuf)],
            [pltpu.VMEM((K, d), jnp.float32) for _ in range(nbuf)],
            pltpu.VMEM_SHARED((n, d), jnp.float32),
            [pltpu.SemaphoreType.DMA for _ in range(nbuf)],
            [pltpu.SemaphoreType.DMA for _ in range(nbuf)],
        ],
    )
    def agg_kernel(table_hbm, src_hbm, dst_hbm, zeros_hbm, out_hbm,
                   srcall, dvs, rvs, acc, gsems, ssems):
        cid, sid, wid = _flat_ids()
        ebase = pl.multiple_of(wid * epw, 8)
        pltpu.sync_copy(src_hbm.at[pl.ds(ebase, epw)], srcall)
        row0 = pl.multiple_of(sid * rpt, 8)

        @pl.when(sid < NS - 1)
        def _():
            pltpu.sync_copy(zeros_hbm.at[pl.ds(row0, rpt)],
                            acc.at[pl.ds(row0, rpt)])

        @pl.when(sid == NS - 1)
        def _():
            pltpu.sync_copy(zeros_hbm.at[pl.ds((NS - 1) * rpt, last)],
                            acc.at[pl.ds((NS - 1) * rpt, last)])

        plsc.subcore_barrier()

        def issue(c, b):
            base = pl.multiple_of(wid * epw + c * K, 8)
            pltpu.async_copy(dst_hbm.at[pl.ds(base, K)], dvs[b], gsems[b])
            sl = pl.ds(pl.multiple_of(c * K, 8), K)
            pltpu.async_copy(table_hbm.at[srcall.at[sl]], rvs[b], gsems[b])

        def wait_gather(b):
            pltpu.make_async_copy(dst_hbm.at[pl.ds(0, K)],
                                  dvs[b], gsems[b]).wait()
            pltpu.make_async_copy(table_hbm.at[pl.ds(0, K)],
                                  rvs[b], gsems[b]).wait()

        def wait_scatter(b):
            pltpu.make_async_copy(table_hbm.at[pl.ds(0, K)],
                                  rvs[b], ssems[b]).wait()

        for b in range(nbuf - 1):
            issue(b, b)

        def body(c, carry):
            for par in range(nbuf):

                @pl.when(lax.rem(c, nbuf) == par)
                def _():
                    wait_gather(par)
                    pltpu.async_copy(rvs[par], acc.at[dvs[par]],
                                     ssems[par], add=True)
                    nb = (par + nbuf - 1) % nbuf

                    @pl.when(c + nbuf - 1 < nchunk)
                    def _():
                        @pl.when(c >= 1)
                        def _():
                            wait_scatter(nb)

                        issue(c + nbuf - 1, nb)

            return carry

        lax.fori_loop(0, nchunk, body, 0)
        for b in range(nbuf):
            wait_scatter(b)
        plsc.subcore_barrier()

        @pl.when(sid < NS - 1)
        def _():
            pltpu.sync_copy(acc.at[pl.ds(row0, rpt)],
                            out_hbm.at[cid, pl.ds(row0, rpt)])

        @pl.when(sid == NS - 1)
        def _():
            pltpu.sync_copy(acc.at[pl.ds((NS - 1) * rpt, last)],
                            out_hbm.at[cid, pl.ds((NS - 1) * rpt, last)])

    return agg_kernel


_R = 1000


def _tc1(degp, x):
    n, d = x.shape

    def body(degp_ref, x_ref, y0_ref, dis_ref):
        deg = jnp.sum(degp_ref[...], axis=0) + 1.0
        dis = lax.rsqrt(deg)[:, None]
        dis_ref[...] = dis
        y0_ref[...] = x_ref[...] * dis

    return pl.pallas_call(
        body,
        out_shape=[
            jax.ShapeDtypeStruct((n, d), jnp.float32),
            jax.ShapeDtypeStruct((n, 1), jnp.float32),
        ],
    )(degp, x)


def _tc2(p, y0, dis, W1, b1):
    n, d = y0.shape
    dh = W1.shape[1]

    def body(p_ref, y0_ref, dis_ref, w_ref, b_ref, ya_ref, yb_ref):
        t = (p_ref[0] + p_ref[1] + y0_ref[...]) * dis_ref[...]
        h = jnp.dot(t, w_ref[...], preferred_element_type=jnp.float32)
        h = jnp.maximum(h + b_ref[...], 0.0)
        y = h * dis_ref[...]
        ya_ref[...] = y[:, :d]
        yb_ref[...] = y[:, d:]

    return pl.pallas_call(
        body,
        grid=(n // _R,),
        in_specs=[
            pl.BlockSpec((NC, _R, d), lambda i: (0, i, 0)),
            pl.BlockSpec((_R, d), lambda i: (i, 0)),
            pl.BlockSpec((_R, 1), lambda i: (i, 0)),
            pl.BlockSpec((d, dh), lambda i: (0, 0)),
            pl.BlockSpec((1, dh), lambda i: (0, 0)),
        ],
        out_specs=[
            pl.BlockSpec((_R, d), lambda i: (i, 0)),
            pl.BlockSpec((_R, d), lambda i: (i, 0)),
        ],
        out_shape=[
            jax.ShapeDtypeStruct((n, d), jnp.float32),
            jax.ShapeDtypeStruct((n, d), jnp.float32),
        ],
    )(p, y0, dis, W1, b1)


def _tc3(pa, pb, y1a, y1b, dis, W2, b2, W3):
    n, d = y1a.shape
    dh = W2.shape[0]

    def body(pa_ref, pb_ref, ya_ref, yb_ref, dis_ref, w2_ref, b2_ref, w3_ref,
             q_ref):
        ta = (pa_ref[0] + pa_ref[1] + ya_ref[...]) * dis_ref[...]
        tb = (pb_ref[0] + pb_ref[1] + yb_ref[...]) * dis_ref[...]
        t = jnp.concatenate([ta, tb], axis=1)
        h2 = jnp.dot(t, w2_ref[...], preferred_element_type=jnp.float32)
        h2 = jnp.maximum(h2 + b2_ref[...], 0.0)
        q = jnp.dot(h2, w3_ref[...], preferred_element_type=jnp.float32)
        q_ref[...] = q * dis_ref[...]

    return pl.pallas_call(
        body,
        grid=(n // _R,),
        in_specs=[
            pl.BlockSpec((NC, _R, d), lambda i: (0, i, 0)),
            pl.BlockSpec((NC, _R, d), lambda i: (0, i, 0)),
            pl.BlockSpec((_R, d), lambda i: (i, 0)),
            pl.BlockSpec((_R, d), lambda i: (i, 0)),
            pl.BlockSpec((_R, 1), lambda i: (i, 0)),
            pl.BlockSpec((dh, dh), lambda i: (0, 0)),
            pl.BlockSpec((1, dh), lambda i: (0, 0)),
            pl.BlockSpec((dh, d), lambda i: (0, 0)),
        ],
        out_specs=pl.BlockSpec((_R, d), lambda i: (i, 0)),
        out_shape=jax.ShapeDtypeStruct((n, d), jnp.float32),
    )(pa, pb, y1a, y1b, dis, W2, b2, W3)


def _tc4(pq, q, dis, b3, Wr, br):
    n, d = q.shape
    do = Wr.shape[1]

    def body(pq_ref, q_ref, dis_ref, b3_ref, wr_ref, br_ref, h_ref, rec_ref):
        z = (pq_ref[0] + pq_ref[1] + q_ref[...]) * dis_ref[...] + b3_ref[...]
        h = jnp.maximum(z, 0.0)
        h_ref[...] = h
        rec = jnp.dot(h, wr_ref[...], preferred_element_type=jnp.float32)
        rec_ref[...] = rec + br_ref[...]

    return pl.pallas_call(
        body,
        grid=(n // _R,),
        in_specs=[
            pl.BlockSpec((NC, _R, d), lambda i: (0, i, 0)),
            pl.BlockSpec((_R, d), lambda i: (i, 0)),
            pl.BlockSpec((_R, 1), lambda i: (i, 0)),
            pl.BlockSpec((1, d), lambda i: (0, 0)),
            pl.BlockSpec((d, do), lambda i: (0, 0)),
            pl.BlockSpec((1, do), lambda i: (0, 0)),
        ],
        out_specs=[
            pl.BlockSpec((_R, d), lambda i: (i, 0)),
            pl.BlockSpec((_R, do), lambda i: (i, 0)),
        ],
        out_shape=[
            jax.ShapeDtypeStruct((n, d), jnp.float32),
            jax.ShapeDtypeStruct((n, do), jnp.float32),
        ],
    )(pq, q, dis, b3, Wr, br)


def kernel(x, edge_index, W1, b1, W2, b2, W3, b3, Wr, br):
    n, d_in = x.shape
    e = edge_index.shape[1]
    src = edge_index[0].astype(jnp.int32)
    dst = edge_index[1].astype(jnp.int32)
    zeros1d = jnp.zeros((n,), jnp.float32)
    zeros2d = jnp.zeros((n, d_in), jnp.float32)

    deg_fn = _make_deg(n, e)
    agg_fn = _make_agg(n, e, d_in)

    degp = deg_fn(dst, zeros1d).reshape(NW, n)
    y0, dis = _tc1(degp, x)
    p1 = agg_fn(y0, src, dst, zeros2d)
    y1a, y1b = _tc2(p1, y0, dis, W1, b1.reshape(1, -1))
    pa = agg_fn(y1a, src, dst, zeros2d)
    pb = agg_fn(y1b, src, dst, zeros2d)
    q = _tc3(pa, pb, y1a, y1b, dis, W2, b2.reshape(1, -1), W3)
    pq = agg_fn(q, src, dst, zeros2d)
    h, rec = _tc4(pq, q, dis, b3.reshape(1, -1), Wr, br.reshape(1, -1))
    return (h, rec)


# speedup vs baseline: 31.0612x; 1.0183x over previous
import functools

import jax
import jax.numpy as jnp
from jax import lax
from jax.experimental import pallas as pl
from jax.experimental.pallas import tpu as pltpu
from jax.experimental.pallas import tpu_sc as plsc

NC, NS = 2, 16
NW = NC * NS
K = 40


def _flat_ids():
    sid = lax.axis_index("s")
    cid = lax.axis_index("c")
    return cid, sid, sid * NC + cid


def _make_deg(n, e):
    epw = e // NW
    chunk = 2000
    nchunk = epw // chunk
    mesh = plsc.VectorSubcoreMesh(core_axis_name="c", subcore_axis_name="s")

    @functools.partial(
        pl.kernel,
        out_type=jax.ShapeDtypeStruct((NW * n,), jnp.float32),
        mesh=mesh,
        scratch_types=[
            pltpu.VMEM((chunk,), jnp.int32),
            pltpu.VMEM((n,), jnp.float32),
        ],
        compiler_params=pltpu.CompilerParams(needs_layout_passes=False),
    )
    def deg_kernel(dst_hbm, zeros_hbm, out_hbm, dstv, degv):
        _, _, wid = _flat_ids()
        pltpu.sync_copy(zeros_hbm, degv)
        ones = jnp.ones((16,), jnp.float32)

        def body(c, carry):
            base = pl.multiple_of(wid * epw + c * chunk, 8)
            pltpu.sync_copy(dst_hbm.at[pl.ds(base, chunk)], dstv)
            for j in range(chunk // 16):
                idx = dstv[pl.ds(j * 16, 16)]
                plsc.addupdate_scatter(degv, [idx], ones)
            return carry

        lax.fori_loop(0, nchunk, body, 0)
        pltpu.sync_copy(degv, out_hbm.at[pl.ds(pl.multiple_of(wid * n, 8), n)])

    return deg_kernel


def _make_agg(n, e, d, ntab=1):
    epw = e // NW
    nchunk = epw // K
    rpt = ((n // NS) // 8) * 8
    last = n - rpt * (NS - 1)
    mesh = plsc.VectorSubcoreMesh(core_axis_name="c", subcore_axis_name="s")

    nbuf = 7
    shape = jax.ShapeDtypeStruct((NC, n, d), jnp.float32)

    @functools.partial(
        pl.kernel,
        out_type=[shape] * ntab if ntab > 1 else shape,
        mesh=mesh,
        scratch_types=[
            pltpu.VMEM((epw,), jnp.int32),
            [pltpu.VMEM((K,), jnp.int32) for _ in range(nbuf)],
            [pltpu.VMEM((K, d), jnp.float32) for _ in range(nbuf)],
            pltpu.VMEM_SHARED((n, d), jnp.float32),
            [pltpu.SemaphoreType.DMA for _ in range(nbuf)],
            [pltpu.SemaphoreType.DMA for _ in range(nbuf)],
        ],
    )
    def agg_kernel(*refs):
        tables = refs[:ntab]
        src_hbm, dst_hbm, zeros_hbm = refs[ntab:ntab + 3]
        outs = refs[ntab + 3:2 * ntab + 3]
        srcall, dvs, rvs, acc, gsems, ssems = refs[2 * ntab + 3:]
        cid, sid, wid = _flat_ids()
        ebase = pl.multiple_of(wid * epw, 8)
        pltpu.sync_copy(src_hbm.at[pl.ds(ebase, epw)], srcall)
        row0 = pl.multiple_of(sid * rpt, 8)

        def zero_acc():
            @pl.when(sid < NS - 1)
            def _():
                pltpu.sync_copy(zeros_hbm.at[pl.ds(row0, rpt)],
                                acc.at[pl.ds(row0, rpt)])

            @pl.when(sid == NS - 1)
            def _():
                pltpu.sync_copy(zeros_hbm.at[pl.ds((NS - 1) * rpt, last)],
                                acc.at[pl.ds((NS - 1) * rpt, last)])

        def writeback(out_hbm):
            @pl.when(sid < NS - 1)
            def _():
                pltpu.sync_copy(acc.at[pl.ds(row0, rpt)],
                                out_hbm.at[cid, pl.ds(row0, rpt)])

            @pl.when(sid == NS - 1)
            def _():
                pltpu.sync_copy(acc.at[pl.ds((NS - 1) * rpt, last)],
                                out_hbm.at[cid, pl.ds((NS - 1) * rpt, last)])

        def wait_gather(b):
            pltpu.make_async_copy(dst_hbm.at[pl.ds(0, K)],
                                  dvs[b], gsems[b]).wait()
            pltpu.make_async_copy(tables[0].at[pl.ds(0, K)],
                                  rvs[b], gsems[b]).wait()

        def wait_scatter(b):
            pltpu.make_async_copy(tables[0].at[pl.ds(0, K)],
                                  rvs[b], ssems[b]).wait()

        def run_pass(table_hbm, out_hbm):
            zero_acc()
            plsc.subcore_barrier()

            def issue(c, b):
                base = pl.multiple_of(wid * epw + c * K, 8)
                pltpu.async_copy(dst_hbm.at[pl.ds(base, K)], dvs[b],
                                 gsems[b])
                sl = pl.ds(pl.multiple_of(c * K, 8), K)
                pltpu.async_copy(table_hbm.at[srcall.at[sl]], rvs[b],
                                 gsems[b])

            for b in range(nbuf - 1):
                issue(b, b)

            def body(c, carry):
                for par in range(nbuf):

                    @pl.when(lax.rem(c, nbuf) == par)
                    def _():
                        wait_gather(par)
                        pltpu.async_copy(rvs[par], acc.at[dvs[par]],
                                         ssems[par], add=True)
                        nb = (par + nbuf - 1) % nbuf

                        @pl.when(c + nbuf - 1 < nchunk)
                        def _():
                            @pl.when(c >= 1)
                            def _():
                                wait_scatter(nb)

                            issue(c + nbuf - 1, nb)

                return carry

            lax.fori_loop(0, nchunk, body, 0)
            for b in range(nbuf):
                wait_scatter(b)
            plsc.subcore_barrier()
            writeback(out_hbm)

        for i in range(ntab):
            run_pass(tables[i], outs[i])

    return agg_kernel


_R = 1000


def _tc1(degp, x):
    n, d = x.shape

    def body(degp_ref, x_ref, y0_ref, dis_ref):
        deg = jnp.sum(degp_ref[...], axis=0) + 1.0
        dis = lax.rsqrt(deg)[:, None]
        dis_ref[...] = dis
        y0_ref[...] = x_ref[...] * dis

    return pl.pallas_call(
        body,
        out_shape=[
            jax.ShapeDtypeStruct((n, d), jnp.float32),
            jax.ShapeDtypeStruct((n, 1), jnp.float32),
        ],
    )(degp, x)


def _tc2(p, y0, dis, W1, b1):
    n, d = y0.shape
    dh = W1.shape[1]

    def body(p_ref, y0_ref, dis_ref, w_ref, b_ref, ya_ref, yb_ref):
        t = (p_ref[0] + p_ref[1] + y0_ref[...]) * dis_ref[...]
        h = jnp.dot(t, w_ref[...], preferred_element_type=jnp.float32)
        h = jnp.maximum(h + b_ref[...], 0.0)
        y = h * dis_ref[...]
        ya_ref[...] = y[:, :d]
        yb_ref[...] = y[:, d:]

    return pl.pallas_call(
        body,
        grid=(n // _R,),
        in_specs=[
            pl.BlockSpec((NC, _R, d), lambda i: (0, i, 0)),
            pl.BlockSpec((_R, d), lambda i: (i, 0)),
            pl.BlockSpec((_R, 1), lambda i: (i, 0)),
            pl.BlockSpec((d, dh), lambda i: (0, 0)),
            pl.BlockSpec((1, dh), lambda i: (0, 0)),
        ],
        out_specs=[
            pl.BlockSpec((_R, d), lambda i: (i, 0)),
            pl.BlockSpec((_R, d), lambda i: (i, 0)),
        ],
        out_shape=[
            jax.ShapeDtypeStruct((n, d), jnp.float32),
            jax.ShapeDtypeStruct((n, d), jnp.float32),
        ],
    )(p, y0, dis, W1, b1)


def _tc3(pa, pb, y1a, y1b, dis, W2, b2, W3):
    n, d = y1a.shape
    dh = W2.shape[0]

    def body(pa_ref, pb_ref, ya_ref, yb_ref, dis_ref, w2_ref, b2_ref, w3_ref,
             q_ref):
        ta = (pa_ref[0] + pa_ref[1] + ya_ref[...]) * dis_ref[...]
        tb = (pb_ref[0] + pb_ref[1] + yb_ref[...]) * dis_ref[...]
        t = jnp.concatenate([ta, tb], axis=1)
        h2 = jnp.dot(t, w2_ref[...], preferred_element_type=jnp.float32)
        h2 = jnp.maximum(h2 + b2_ref[...], 0.0)
        q = jnp.dot(h2, w3_ref[...], preferred_element_type=jnp.float32)
        q_ref[...] = q * dis_ref[...]

    return pl.pallas_call(
        body,
        grid=(n // _R,),
        in_specs=[
            pl.BlockSpec((NC, _R, d), lambda i: (0, i, 0)),
            pl.BlockSpec((NC, _R, d), lambda i: (0, i, 0)),
            pl.BlockSpec((_R, d), lambda i: (i, 0)),
            pl.BlockSpec((_R, d), lambda i: (i, 0)),
            pl.BlockSpec((_R, 1), lambda i: (i, 0)),
            pl.BlockSpec((dh, dh), lambda i: (0, 0)),
            pl.BlockSpec((1, dh), lambda i: (0, 0)),
            pl.BlockSpec((dh, d), lambda i: (0, 0)),
        ],
        out_specs=pl.BlockSpec((_R, d), lambda i: (i, 0)),
        out_shape=jax.ShapeDtypeStruct((n, d), jnp.float32),
    )(pa, pb, y1a, y1b, dis, W2, b2, W3)


def _tc4(pq, q, dis, b3, Wr, br):
    n, d = q.shape
    do = Wr.shape[1]

    def body(pq_ref, q_ref, dis_ref, b3_ref, wr_ref, br_ref, h_ref, rec_ref):
        z = (pq_ref[0] + pq_ref[1] + q_ref[...]) * dis_ref[...] + b3_ref[...]
        h = jnp.maximum(z, 0.0)
        h_ref[...] = h
        rec = jnp.dot(h, wr_ref[...], preferred_element_type=jnp.float32)
        rec_ref[...] = rec + br_ref[...]

    return pl.pallas_call(
        body,
        grid=(n // _R,),
        in_specs=[
            pl.BlockSpec((NC, _R, d), lambda i: (0, i, 0)),
            pl.BlockSpec((_R, d), lambda i: (i, 0)),
            pl.BlockSpec((_R, 1), lambda i: (i, 0)),
            pl.BlockSpec((1, d), lambda i: (0, 0)),
            pl.BlockSpec((d, do), lambda i: (0, 0)),
            pl.BlockSpec((1, do), lambda i: (0, 0)),
        ],
        out_specs=[
            pl.BlockSpec((_R, d), lambda i: (i, 0)),
            pl.BlockSpec((_R, do), lambda i: (i, 0)),
        ],
        out_shape=[
            jax.ShapeDtypeStruct((n, d), jnp.float32),
            jax.ShapeDtypeStruct((n, do), jnp.float32),
        ],
    )(pq, q, dis, b3, Wr, br)


def kernel(x, edge_index, W1, b1, W2, b2, W3, b3, Wr, br):
    n, d_in = x.shape
    e = edge_index.shape[1]
    src = edge_index[0].astype(jnp.int32)
    dst = edge_index[1].astype(jnp.int32)
    zeros1d = jnp.zeros((n,), jnp.float32)
    zeros2d = jnp.zeros((n, d_in), jnp.float32)

    deg_fn = _make_deg(n, e)
    agg_fn = _make_agg(n, e, d_in)
    agg2_fn = _make_agg(n, e, d_in, ntab=2)

    degp = deg_fn(dst, zeros1d).reshape(NW, n)
    y0, dis = _tc1(degp, x)
    p1 = agg_fn(y0, src, dst, zeros2d)
    y1a, y1b = _tc2(p1, y0, dis, W1, b1.reshape(1, -1))
    pa, pb = agg2_fn(y1a, y1b, src, dst, zeros2d)
    q = _tc3(pa, pb, y1a, y1b, dis, W2, b2.reshape(1, -1), W3)
    pq = agg_fn(q, src, dst, zeros2d)
    h, rec = _tc4(pq, q, dis, b3.reshape(1, -1), Wr, br.reshape(1, -1))
    return (h, rec)
